# Initial kernel scaffold; baseline (speedup 1.0000x reference)
import jax
import jax.numpy as jnp
from jax.experimental import pallas as pl


def kernel(logits, labels):
    raise NotImplementedError("write your pallas kernel here")


# speedup vs baseline: 14.4556x; 14.4556x over previous
import functools

import jax
import jax.numpy as jnp
from jax import lax
from jax.experimental import pallas as pl
from jax.experimental.pallas import tpu as pltpu
from jax.experimental.pallas import tpu_sc as plsc

_THRESH = 0.3
_IGNORE = 255
_K = 131072
_C = 19
_BN, _H, _W = 8, 512, 512
_NPIX = _BN * _H * _W

_BH = 128

_NC, _NS, _L = 2, 16, 16
_NW = _NC * _NS
_PER_W = _NPIX // _NW
_CHUNK = 8192
_NCHUNK = _PER_W // _CHUNK

_HB = 512
_SCALE = _HB / _THRESH


def _ce_stage1(logits_ref, labels_ref, loss_ref, cnt_ref, summask_ref):
    x = logits_ref[0]
    lbl = labels_ref[0]
    m = jnp.max(x, axis=0)
    lse = jnp.log(jnp.sum(jnp.exp(x - m[None]), axis=0)) + m
    cls = lax.broadcasted_iota(jnp.int32, x.shape, 0)
    picked = jnp.sum(jnp.where(cls == lbl[None], x, 0.0), axis=0)
    loss = jnp.where(lbl != _IGNORE, lse - picked, 0.0)
    loss_ref[0] = loss

    msk = loss > _THRESH
    bc = jnp.sum(msk.astype(jnp.float32))
    bs = jnp.sum(jnp.where(msk, loss, 0.0))

    @pl.when((pl.program_id(0) == 0) & (pl.program_id(1) == 0))
    def _init():
        cnt_ref[0, 0] = 0.0
        summask_ref[0, 0] = 0.0

    cnt_ref[0, 0] += bc
    summask_ref[0, 0] += bs


def _stage1(logits, labels):
    return pl.pallas_call(
        _ce_stage1,
        grid=(_BN, _H // _BH),
        in_specs=[
            pl.BlockSpec((1, _C, _BH, _W), lambda i, j: (i, 0, j, 0)),
            pl.BlockSpec((1, _BH, _W), lambda i, j: (i, j, 0)),
        ],
        out_specs=[
            pl.BlockSpec((1, _BH, _W), lambda i, j: (i, j, 0)),
            pl.BlockSpec(memory_space=pltpu.SMEM),
            pl.BlockSpec(memory_space=pltpu.SMEM),
        ],
        out_shape=[
            jax.ShapeDtypeStruct((_BN, _H, _W), jnp.float32),
            jax.ShapeDtypeStruct((1, 1), jnp.float32),
            jax.ShapeDtypeStruct((1, 1), jnp.float32),
        ],
    )(logits, labels)


def _sc_hist(flat_losses):
    mesh = plsc.VectorSubcoreMesh(core_axis_name="c", subcore_axis_name="s")

    @functools.partial(
        pl.kernel,
        mesh=mesh,
        compiler_params=pltpu.CompilerParams(needs_layout_passes=False),
        out_type=jax.ShapeDtypeStruct((_NW, 2 * _HB), jnp.float32),
        scratch_types=[
            pltpu.VMEM((_CHUNK,), jnp.float32),
            pltpu.VMEM((_L * _HB,), jnp.float32),
            pltpu.VMEM((_L * _HB,), jnp.float32),
            pltpu.VMEM((2 * _HB,), jnp.float32),
        ],
    )
    def hist_kernel(loss_hbm, out_hbm, chunk_v, hcnt_v, hsum_v, outrow_v):
        wid = lax.axis_index("s") * _NC + lax.axis_index("c")
        lane = lax.iota(jnp.int32, 16)
        ones = jnp.ones((_L,), jnp.float32)
        zeros = jnp.zeros((_L,), jnp.float32)

        def zero_body(i, carry):
            hcnt_v[pl.ds(i * _L, _L)] = zeros
            hsum_v[pl.ds(i * _L, _L)] = zeros
            return carry

        lax.fori_loop(0, _HB, zero_body, 0)

        base = wid * _PER_W

        def chunk_body(c, carry):
            pltpu.sync_copy(loss_hbm.at[pl.ds(base + c * _CHUNK, _CHUNK)],
                            chunk_v)

            def vec_body(i, carry2):
                v = chunk_v[pl.ds(i * _L, _L)]
                msk = v <= _THRESH
                idx = jnp.clip((jnp.minimum(v, _THRESH) * _SCALE)
                               .astype(jnp.int32), 0, _HB - 1)
                fidx = lane * _HB + idx
                plsc.addupdate_scatter(hcnt_v, [fidx], ones, mask=msk)
                plsc.addupdate_scatter(hsum_v, [fidx], v, mask=msk)
                return carry2

            lax.fori_loop(0, _CHUNK // _L, vec_body, 0)
            return carry

        lax.fori_loop(0, _NCHUNK, chunk_body, 0)

        def red_body(c, carry):
            acc_c = zeros
            acc_s = zeros
            for r in range(_L):
                acc_c = acc_c + hcnt_v[pl.ds(r * _HB + c * _L, _L)]
                acc_s = acc_s + hsum_v[pl.ds(r * _HB + c * _L, _L)]
            outrow_v[pl.ds(c * _L, _L)] = acc_c
            outrow_v[pl.ds(_HB + c * _L, _L)] = acc_s
            return carry

        lax.fori_loop(0, _HB // _L, red_body, 0)
        pltpu.sync_copy(outrow_v, out_hbm.at[wid])

    return hist_kernel(flat_losses)


def _combine(cnt, summask, hcnt, hsum):
    rc_cnt = hcnt[::-1]
    rc_sum = hsum[::-1]
    cum_c = jnp.cumsum(rc_cnt)
    prev_c = cum_c - rc_cnt
    need = jnp.maximum(_K - cnt, 0.0)
    take = jnp.clip(need - prev_c, 0.0, rc_cnt)
    mean_b = rc_sum / jnp.maximum(rc_cnt, 1.0)
    contrib = jnp.where(take >= rc_cnt, rc_sum, take * mean_b)
    topk_mean = (summask + jnp.sum(contrib)) / _K
    masked_mean = summask / jnp.maximum(cnt, 1.0)
    return jnp.where(cnt > _K, masked_mean, topk_mean)


def kernel(logits, labels):
    losses, cnt, summask = _stage1(logits, labels)
    cnt = cnt[0, 0]
    summask = summask[0, 0]
    rows = _sc_hist(losses.reshape(-1))
    hist = rows.reshape(_NW, 2, _HB).sum(axis=0)
    return _combine(cnt, summask, hist[0], hist[1])
